# baseline (speedup 1.0000x reference)
import functools

import jax
import jax.numpy as jnp
from jax.experimental import pallas as pl

Q = 16384
K = 1000
D = 16
KP = 1024
BQ = 1024
GRID = Q // BQ


def _body(x_ref, ct_ref, out_ref):
    x = x_ref[...]
    ct = ct_ref[...]
    x2 = jnp.sum(x * x, axis=1, keepdims=True)
    c2 = jnp.sum(ct * ct, axis=0, keepdims=True)
    mm = jnp.dot(x, ct, preferred_element_type=jnp.float32)
    d2 = jnp.maximum(x2 + c2 - 2.0 * mm, 0.0)
    m = jnp.min(d2, axis=1, keepdims=True)
    iota = jax.lax.broadcasted_iota(jnp.int32, (BQ, KP), 1)
    idx = jnp.min(jnp.where(d2 == m, iota, KP), axis=1)
    out_ref[0, 0, :] = idx


@jax.jit
def kernel(features, cluster_centers):
    pad = jnp.full((KP - K, D), 1e17, dtype=cluster_centers.dtype)
    ct = jnp.concatenate([cluster_centers, pad], axis=0).T

    out = pl.pallas_call(
        _body,
        grid=(GRID,),
        in_specs=[
            pl.BlockSpec((BQ, D), lambda i: (i, 0)),
            pl.BlockSpec((D, KP), lambda i: (0, 0)),
        ],
        out_specs=pl.BlockSpec((1, 1, BQ), lambda i: (i, 0, 0)),
        out_shape=jax.ShapeDtypeStruct((GRID, 1, BQ), jnp.int32),
    )(features, ct)
    return out.reshape(Q)


# speedup vs baseline: 1.0741x; 1.0741x over previous
import jax
import jax.numpy as jnp
from jax.experimental import pallas as pl
from jax.experimental.pallas import tpu as pltpu

Q = 16384
K = 1000
D = 16
KP = 1024
BQ = 1024
GRID = Q // BQ


def _body(x_ref, ct_ref, out_ref, c2_ref, x2_ref, mm_ref):
    i = pl.program_id(0)

    @pl.when(i == 0)
    def _prep():
        ct = ct_ref[...]
        c2_ref[...] = jnp.sum(ct * ct, axis=0, keepdims=True)

    @pl.when(i < GRID)
    def _mxu():
        x = x_ref[...]
        x2_ref[i % 2] = jnp.sum(x * x, axis=1, keepdims=True)
        mm_ref[i % 2] = jnp.dot(x, ct_ref[...],
                                preferred_element_type=jnp.float32)

    @pl.when(i > 0)
    def _argmin():
        mm = mm_ref[(i - 1) % 2]
        d2 = (x2_ref[(i - 1) % 2] + c2_ref[...]) - 2.0 * mm
        d2 = jnp.maximum(d2, 0.0)
        m = jnp.min(d2, axis=1, keepdims=True)
        iota = jax.lax.broadcasted_iota(jnp.int32, (BQ, KP), 1)
        out_ref[0, 0, :] = jnp.min(jnp.where(d2 == m, iota, KP), axis=1)


@jax.jit
def kernel(features, cluster_centers):
    pad = jnp.full((KP - K, D), 1e17, dtype=cluster_centers.dtype)
    ct = jnp.concatenate([cluster_centers, pad], axis=0).T

    out = pl.pallas_call(
        _body,
        grid=(GRID + 1,),
        in_specs=[
            pl.BlockSpec((BQ, D), lambda i: (jnp.minimum(i, GRID - 1), 0)),
            pl.BlockSpec((D, KP), lambda i: (0, 0)),
        ],
        out_specs=pl.BlockSpec((1, 1, BQ), lambda i: (jnp.maximum(i - 1, 0), 0, 0)),
        out_shape=jax.ShapeDtypeStruct((GRID, 1, BQ), jnp.int32),
        scratch_shapes=[
            pltpu.VMEM((1, KP), jnp.float32),
            pltpu.VMEM((2, BQ, 1), jnp.float32),
            pltpu.VMEM((2, BQ, KP), jnp.float32),
        ],
    )(features, ct)
    return out.reshape(Q)


# speedup vs baseline: 1.4245x; 1.3261x over previous
import jax
import jax.numpy as jnp
from jax.experimental import pallas as pl
from jax.experimental.pallas import tpu as pltpu

Q = 16384
K = 1000
D = 16
KP = 1024
BQ = 1024
GRID = Q // BQ


def _body(c_ref, xt_ref, out_ref, c2_ref, x2_ref, mm_ref):
    i = pl.program_id(0)

    @pl.when(i == 0)
    def _prep():
        c = c_ref[...]
        c2_ref[...] = jnp.sum(c * c, axis=1, keepdims=True)

    @pl.when(i < GRID)
    def _mxu():
        xt = xt_ref[...]
        x2_ref[i % 2] = jnp.sum(xt * xt, axis=0, keepdims=True)
        mm_ref[i % 2] = jnp.dot(c_ref[...], xt,
                                preferred_element_type=jnp.float32)

    @pl.when(i > 0)
    def _argmin():
        mm = mm_ref[(i - 1) % 2]
        d2 = (x2_ref[(i - 1) % 2] + c2_ref[...]) - 2.0 * mm
        d2 = jnp.maximum(d2, 0.0)
        m = jnp.min(d2, axis=0, keepdims=True)
        iota = jax.lax.broadcasted_iota(jnp.int32, (KP, BQ), 0)
        out_ref[0, 0, :] = jnp.min(jnp.where(d2 == m, iota, KP), axis=0)


@jax.jit
def kernel(features, cluster_centers):
    pad = jnp.full((KP - K, D), 1e17, dtype=cluster_centers.dtype)
    c = jnp.concatenate([cluster_centers, pad], axis=0)
    xt = features.T

    out = pl.pallas_call(
        _body,
        grid=(GRID + 1,),
        in_specs=[
            pl.BlockSpec((KP, D), lambda i: (0, 0)),
            pl.BlockSpec((D, BQ), lambda i: (0, jnp.minimum(i, GRID - 1))),
        ],
        out_specs=pl.BlockSpec((1, 1, BQ), lambda i: (jnp.maximum(i - 1, 0), 0, 0)),
        out_shape=jax.ShapeDtypeStruct((GRID, 1, BQ), jnp.int32),
        scratch_shapes=[
            pltpu.VMEM((KP, 1), jnp.float32),
            pltpu.VMEM((2, 1, BQ), jnp.float32),
            pltpu.VMEM((2, KP, BQ), jnp.float32),
        ],
    )(c, xt)
    return out.reshape(Q)
